# baseline (speedup 1.0000x reference)
import jax
import jax.numpy as jnp
from jax import lax
from jax.experimental import pallas as pl
from jax.experimental.pallas import tpu as pltpu
from jax.experimental.pallas import tpu_sc as plsc

NUM_CORES = 2
NUM_SUBCORES = 16
NW = NUM_CORES * NUM_SUBCORES
B = 16384
D = 32
NU = 1000000
NI = 100000
NBU = NU // 128
NBI = NI // 128
BPW = B // NW


NB2U = NU // 256


def _detile_body(ut, it, tailu, taili, upk, ipk,
                 inb_a, inb_b, ob_a, ob_b, s_ia, s_ib, s_oa, s_ob):
    wid = lax.axis_index("s") * NUM_CORES + lax.axis_index("c")
    iota = jnp.arange(16, dtype=jnp.int32)
    row_ch = [((16 * ch + iota) >> 2) for ch in range(8)]
    colb_ch = [(32 * ((16 * ch + iota) & 3)) for ch in range(8)]

    bufs = ((inb_a, ob_a, s_ia, s_oa), (inb_b, ob_b, s_ib, s_ob))

    def transpose_sb(inb, ob, nsb):
        def bloop(b, _):
            def qloop(q, _):
                rot = (iota + q) & 15
                rsh = rot >> 2
                cm3 = 32 * (rot & 3)
                for e in range(2):
                    vs = [plsc.load_gather(
                        inb, [iota + 16 * e, 128 * b + 16 * p + rot])
                        for p in range(8)]
                    for p in range(8):
                        plsc.store_scatter(
                            ob, [32 * b + 4 * p + rsh, cm3 + 16 * e + iota],
                            vs[p])
                return 0
            lax.fori_loop(0, 16, qloop, 0)
            return 0
        lax.fori_loop(0, nsb, bloop, 0)

    def ring(tab, out, nb2, nsb, npair):
        cw = 512 * nsb
        orows = 32 * nsb

        def fire_in(j2, inb, sem):
            @pl.when(j2 < nb2)
            def _():
                off = pl.multiple_of(j2 * cw, 128)
                pltpu.async_copy(tab.at[:, pl.ds(off, cw)],
                                 inb.at[:, pl.ds(0, cw)], sem)

        def wait_in(j2, inb, sem):
            @pl.when(j2 < nb2)
            def _():
                pltpu.make_async_copy(tab.at[:, pl.ds(0, cw)],
                                      inb.at[:, pl.ds(0, cw)], sem).wait()

        def fire_out(j2, ob, sem):
            @pl.when(j2 < nb2)
            def _():
                off = pl.multiple_of(j2 * orows, 8)
                pltpu.async_copy(ob.at[pl.ds(0, orows), :],
                                 out.at[pl.ds(off, orows), :], sem)

        def wait_out(j2, ob, sem):
            @pl.when(j2 < nb2)
            def _():
                pltpu.make_async_copy(ob.at[pl.ds(0, orows), :],
                                      out.at[pl.ds(0, orows), :], sem).wait()

        def pair(t, _):
            for i, (inb, ob, s_i, s_o) in enumerate(bufs):
                j2 = wid + (2 * t + i) * NW
                wait_in(j2, inb, s_i)

                @pl.when((t > 0) & (j2 < nb2))
                def _():
                    pltpu.make_async_copy(
                        ob.at[pl.ds(0, orows), :],
                        out.at[pl.ds(0, orows), :], s_o).wait()

                @pl.when(j2 < nb2)
                def _():
                    transpose_sb(inb, ob, nsb)
                fire_out(j2, ob, s_o)
                fire_in(j2 + 2 * NW, inb, s_i)
            return 0

        for i, (inb, ob, s_i, s_o) in enumerate(bufs):
            fire_in(wid + i * NW, inb, s_i)
        lax.fori_loop(0, npair, pair, 0)
        for i, (inb, ob, s_i, s_o) in enumerate(bufs):
            wait_in(wid + (2 * npair + i) * NW, inb, s_i)
        for i, (inb, ob, s_i, s_o) in enumerate(bufs):
            wait_out(wid + i * NW, ob, s_o)

    ring(ut, upk, NB2U, 2, 62)
    ring(it, ipk, NBI, 1, 13)

    @pl.when(wid == 0)
    def _():
        pltpu.sync_copy(tailu, ob_a.at[pl.ds(0, 16), :])
        pltpu.sync_copy(ob_a.at[pl.ds(0, 16), :],
                        upk.at[pl.ds(NBU * 32, 16), :])

    @pl.when(wid == 1)
    def _():
        pltpu.sync_copy(taili, ob_a.at[pl.ds(0, 8), :])
        pltpu.sync_copy(ob_a.at[pl.ds(0, 8), :],
                        ipk.at[pl.ds(NBI * 32, 8), :])


_detile = pl.kernel(
    _detile_body,
    mesh=plsc.VectorSubcoreMesh(core_axis_name="c", subcore_axis_name="s"),
    out_type=[
        jax.ShapeDtypeStruct((NU // 4, 128), jnp.float32),
        jax.ShapeDtypeStruct((NI // 4, 128), jnp.float32),
    ],
    scratch_types=[
        pltpu.VMEM((32, 256), jnp.float32),
        pltpu.VMEM((32, 256), jnp.float32),
        pltpu.VMEM((64, 128), jnp.float32),
        pltpu.VMEM((64, 128), jnp.float32),
        pltpu.SemaphoreType.DMA,
        pltpu.SemaphoreType.DMA,
        pltpu.SemaphoreType.DMA,
        pltpu.SemaphoreType.DMA,
    ],
    compiler_params=pltpu.CompilerParams(needs_layout_passes=False),
)


def _gather_body(upk, ipk, uidx, iidx, uvf, ivf,
                 idxr_v, ridx_v, sub_v, rows_v, comp_v, sem):
    wid = lax.axis_index("s") * NUM_CORES + lax.axis_index("c")
    base = wid * BPW
    iota = jnp.arange(16, dtype=jnp.int32)

    for idx, src, outf in ((uidx, upk, uvf), (iidx, ipk, ivf)):
        pltpu.sync_copy(idx.at[pl.ds(base, BPW)], idxr_v)
        for k in range(BPW // 16):
            v = idxr_v[pl.ds(16 * k, 16)]
            ridx_v[pl.ds(16 * k, 16)] = v >> 2
            sub_v[pl.ds(16 * k, 16)] = v & 3
        descs = []
        for jj in range(BPW // 128):
            descs.append(pltpu.async_copy(
                src.at[ridx_v.at[pl.ds(jj * 128, 128)]],
                rows_v.at[pl.ds(jj * 128, 128)], sem))
        for dsc in descs:
            dsc.wait()

        def ext(i, _):
            si = jnp.full((16,), i, jnp.int32)
            s16 = plsc.load_gather(sub_v, [si])
            cbase = s16 * 32
            g1 = plsc.load_gather(rows_v, [si, cbase + iota])
            g2 = plsc.load_gather(rows_v, [si, cbase + iota + 16])
            ob = i * 32
            plsc.store_scatter(comp_v, [ob + iota], g1)
            plsc.store_scatter(comp_v, [ob + 16 + iota], g2)
            return 0

        lax.fori_loop(0, BPW, ext, 0)
        pltpu.sync_copy(comp_v, outf.at[pl.ds(base * D, BPW * D)])


_gather = pl.kernel(
    _gather_body,
    mesh=plsc.VectorSubcoreMesh(core_axis_name="c", subcore_axis_name="s"),
    out_type=[
        jax.ShapeDtypeStruct((B * D,), jnp.float32),
        jax.ShapeDtypeStruct((B * D,), jnp.float32),
    ],
    scratch_types=[
        pltpu.VMEM((BPW,), jnp.int32),
        pltpu.VMEM((BPW,), jnp.int32),
        pltpu.VMEM((BPW,), jnp.int32),
        pltpu.VMEM((BPW, 128), jnp.float32),
        pltpu.VMEM((BPW * D,), jnp.float32),
        pltpu.SemaphoreType.DMA,
    ],
    compiler_params=pltpu.CompilerParams(needs_layout_passes=False),
)


BLK = 2048


def _mlp_body(uv, iv, w1u, w1i, b1, w2t, b2, w3, b3, out):
    h = jnp.dot(uv[...], w1u[...], preferred_element_type=jnp.float32)
    h = h + jnp.dot(iv[...], w1i[...], preferred_element_type=jnp.float32)
    h = jnp.maximum(h + b1[...], 0.0)
    h2 = jnp.dot(h, w2t[...], preferred_element_type=jnp.float32) + b2[...]
    h2 = jnp.maximum(h2, 0.0)
    out[...] = jnp.sum(h2 * w3[...], axis=1) + b3[0, 0]


def _mlp(uv, iv, w1u, w1i, b1, w2t, b2, w3, b3):
    return pl.pallas_call(
        _mlp_body,
        grid=(B // BLK,),
        in_specs=[
            pl.BlockSpec((BLK, D), lambda i: (i, 0)),
            pl.BlockSpec((BLK, D), lambda i: (i, 0)),
            pl.BlockSpec((D, 64), lambda i: (0, 0)),
            pl.BlockSpec((D, 64), lambda i: (0, 0)),
            pl.BlockSpec((1, 64), lambda i: (0, 0)),
            pl.BlockSpec((64, 32), lambda i: (0, 0)),
            pl.BlockSpec((1, 32), lambda i: (0, 0)),
            pl.BlockSpec((1, 32), lambda i: (0, 0)),
            pl.BlockSpec((1, 1), lambda i: (0, 0), memory_space=pltpu.SMEM),
        ],
        out_specs=pl.BlockSpec((BLK,), lambda i: (i,)),
        out_shape=jax.ShapeDtypeStruct((B,), jnp.float32),
    )(uv, iv, w1u, w1i, b1, w2t, b2, w3, b3)


def kernel(user_indices, item_indices, user_table, item_table,
           W1, b1, W2, b2, W3, b3):
    tailu = user_table[NBU * 128:].reshape(16, 128)
    taili = item_table[NBI * 128:].reshape(8, 128)
    upk, ipk = _detile(user_table.T, item_table.T, tailu, taili)
    uvf, ivf = _gather(upk, ipk,
                       user_indices.astype(jnp.int32),
                       item_indices.astype(jnp.int32))
    uv = uvf.reshape(B, D)
    iv = ivf.reshape(B, D)
    w1u = W1[:, :D].T
    w1i = W1[:, D:].T
    return _mlp(uv, iv, w1u, w1i, b1.reshape(1, 64), W2.T,
                b2.reshape(1, 32), W3, b3.reshape(1, 1))


# speedup vs baseline: 1.0299x; 1.0299x over previous
import jax
import jax.numpy as jnp
from jax import lax
from jax.experimental import pallas as pl
from jax.experimental.pallas import tpu as pltpu
from jax.experimental.pallas import tpu_sc as plsc

NUM_CORES = 2
NUM_SUBCORES = 16
NW = NUM_CORES * NUM_SUBCORES
B = 16384
D = 32
NU = 1000000
NI = 100000
NBU = NU // 128
NBI = NI // 128
BPW = B // NW


NB2U = NU // 512


def _detile_body(ut, it, tailu, taili, upk, ipk,
                 inb_a, inb_b, ob_a, ob_b, s_ia, s_ib, s_oa, s_ob):
    wid = lax.axis_index("s") * NUM_CORES + lax.axis_index("c")
    iota = jnp.arange(16, dtype=jnp.int32)
    row_ch = [((16 * ch + iota) >> 2) for ch in range(8)]
    colb_ch = [(32 * ((16 * ch + iota) & 3)) for ch in range(8)]

    bufs = ((inb_a, ob_a, s_ia, s_oa), (inb_b, ob_b, s_ib, s_ob))

    def transpose_sb(inb, ob, nsb):
        def bloop(b, _):
            def qloop(q, _):
                rot = (iota + q) & 15
                rsh = rot >> 2
                cm3 = 32 * (rot & 3)
                for e in range(2):
                    vs = [plsc.load_gather(
                        inb, [iota + 16 * e, 128 * b + 16 * p + rot])
                        for p in range(8)]
                    for p in range(8):
                        plsc.store_scatter(
                            ob, [32 * b + 4 * p + rsh, cm3 + 16 * e + iota],
                            vs[p])
                return 0
            lax.fori_loop(0, 16, qloop, 0)
            return 0
        lax.fori_loop(0, nsb, bloop, 0)

    def ring(tab, out, nb2, nsb, npair):
        cw = 512 * nsb
        orows = 32 * nsb

        def fire_in(j2, inb, sem):
            @pl.when(j2 < nb2)
            def _():
                off = pl.multiple_of(j2 * cw, 128)
                pltpu.async_copy(tab.at[:, pl.ds(off, cw)],
                                 inb.at[:, pl.ds(0, cw)], sem)

        def wait_in(j2, inb, sem):
            @pl.when(j2 < nb2)
            def _():
                pltpu.make_async_copy(tab.at[:, pl.ds(0, cw)],
                                      inb.at[:, pl.ds(0, cw)], sem).wait()

        def fire_out(j2, ob, sem):
            @pl.when(j2 < nb2)
            def _():
                off = pl.multiple_of(j2 * orows, 8)
                pltpu.async_copy(ob.at[pl.ds(0, orows), :],
                                 out.at[pl.ds(off, orows), :], sem)

        def wait_out(j2, ob, sem):
            @pl.when(j2 < nb2)
            def _():
                pltpu.make_async_copy(ob.at[pl.ds(0, orows), :],
                                      out.at[pl.ds(0, orows), :], sem).wait()

        def pair(t, _):
            for i, (inb, ob, s_i, s_o) in enumerate(bufs):
                j2 = wid + (2 * t + i) * NW
                wait_in(j2, inb, s_i)

                @pl.when((t > 0) & (j2 < nb2))
                def _():
                    pltpu.make_async_copy(
                        ob.at[pl.ds(0, orows), :],
                        out.at[pl.ds(0, orows), :], s_o).wait()

                @pl.when(j2 < nb2)
                def _():
                    transpose_sb(inb, ob, nsb)
                fire_out(j2, ob, s_o)
                fire_in(j2 + 2 * NW, inb, s_i)
            return 0

        for i, (inb, ob, s_i, s_o) in enumerate(bufs):
            fire_in(wid + i * NW, inb, s_i)
        lax.fori_loop(0, npair, pair, 0)
        for i, (inb, ob, s_i, s_o) in enumerate(bufs):
            wait_in(wid + (2 * npair + i) * NW, inb, s_i)
        for i, (inb, ob, s_i, s_o) in enumerate(bufs):
            wait_out(wid + i * NW, ob, s_o)

    ring(ut, upk, NB2U, 4, 31)
    ring(it, ipk, NBI, 1, 13)

    @pl.when(wid == 0)
    def _():
        pltpu.sync_copy(tailu, ob_a.at[pl.ds(0, 16), :])
        pltpu.sync_copy(ob_a.at[pl.ds(0, 16), :],
                        upk.at[pl.ds(NBU * 32, 16), :])

    @pl.when(wid == 1)
    def _():
        pltpu.sync_copy(taili, ob_a.at[pl.ds(0, 8), :])
        pltpu.sync_copy(ob_a.at[pl.ds(0, 8), :],
                        ipk.at[pl.ds(NBI * 32, 8), :])


_detile = pl.kernel(
    _detile_body,
    mesh=plsc.VectorSubcoreMesh(core_axis_name="c", subcore_axis_name="s"),
    out_type=[
        jax.ShapeDtypeStruct((NU // 4, 128), jnp.float32),
        jax.ShapeDtypeStruct((NI // 4, 128), jnp.float32),
    ],
    scratch_types=[
        pltpu.VMEM((32, 512), jnp.float32),
        pltpu.VMEM((32, 512), jnp.float32),
        pltpu.VMEM((128, 128), jnp.float32),
        pltpu.VMEM((128, 128), jnp.float32),
        pltpu.SemaphoreType.DMA,
        pltpu.SemaphoreType.DMA,
        pltpu.SemaphoreType.DMA,
        pltpu.SemaphoreType.DMA,
    ],
    compiler_params=pltpu.CompilerParams(needs_layout_passes=False),
)


def _gather_body(upk, ipk, uidx, iidx, uvf, ivf,
                 idxr_v, ridx_v, sub_v, rows_v, comp_v, sem):
    wid = lax.axis_index("s") * NUM_CORES + lax.axis_index("c")
    base = wid * BPW
    iota = jnp.arange(16, dtype=jnp.int32)

    for idx, src, outf in ((uidx, upk, uvf), (iidx, ipk, ivf)):
        pltpu.sync_copy(idx.at[pl.ds(base, BPW)], idxr_v)
        for k in range(BPW // 16):
            v = idxr_v[pl.ds(16 * k, 16)]
            ridx_v[pl.ds(16 * k, 16)] = v >> 2
            sub_v[pl.ds(16 * k, 16)] = v & 3
        descs = []
        for jj in range(BPW // 128):
            descs.append(pltpu.async_copy(
                src.at[ridx_v.at[pl.ds(jj * 128, 128)]],
                rows_v.at[pl.ds(jj * 128, 128)], sem))
        for dsc in descs:
            dsc.wait()

        def ext(i, _):
            si = jnp.full((16,), i, jnp.int32)
            s16 = plsc.load_gather(sub_v, [si])
            cbase = s16 * 32
            g1 = plsc.load_gather(rows_v, [si, cbase + iota])
            g2 = plsc.load_gather(rows_v, [si, cbase + iota + 16])
            ob = i * 32
            plsc.store_scatter(comp_v, [ob + iota], g1)
            plsc.store_scatter(comp_v, [ob + 16 + iota], g2)
            return 0

        lax.fori_loop(0, BPW, ext, 0)
        pltpu.sync_copy(comp_v, outf.at[pl.ds(base * D, BPW * D)])


_gather = pl.kernel(
    _gather_body,
    mesh=plsc.VectorSubcoreMesh(core_axis_name="c", subcore_axis_name="s"),
    out_type=[
        jax.ShapeDtypeStruct((B * D,), jnp.float32),
        jax.ShapeDtypeStruct((B * D,), jnp.float32),
    ],
    scratch_types=[
        pltpu.VMEM((BPW,), jnp.int32),
        pltpu.VMEM((BPW,), jnp.int32),
        pltpu.VMEM((BPW,), jnp.int32),
        pltpu.VMEM((BPW, 128), jnp.float32),
        pltpu.VMEM((BPW * D,), jnp.float32),
        pltpu.SemaphoreType.DMA,
    ],
    compiler_params=pltpu.CompilerParams(needs_layout_passes=False),
)


BLK = 2048


def _mlp_body(uv, iv, w1u, w1i, b1, w2t, b2, w3, b3, out):
    h = jnp.dot(uv[...], w1u[...], preferred_element_type=jnp.float32)
    h = h + jnp.dot(iv[...], w1i[...], preferred_element_type=jnp.float32)
    h = jnp.maximum(h + b1[...], 0.0)
    h2 = jnp.dot(h, w2t[...], preferred_element_type=jnp.float32) + b2[...]
    h2 = jnp.maximum(h2, 0.0)
    out[...] = jnp.sum(h2 * w3[...], axis=1) + b3[0, 0]


def _mlp(uv, iv, w1u, w1i, b1, w2t, b2, w3, b3):
    return pl.pallas_call(
        _mlp_body,
        grid=(B // BLK,),
        in_specs=[
            pl.BlockSpec((BLK, D), lambda i: (i, 0)),
            pl.BlockSpec((BLK, D), lambda i: (i, 0)),
            pl.BlockSpec((D, 64), lambda i: (0, 0)),
            pl.BlockSpec((D, 64), lambda i: (0, 0)),
            pl.BlockSpec((1, 64), lambda i: (0, 0)),
            pl.BlockSpec((64, 32), lambda i: (0, 0)),
            pl.BlockSpec((1, 32), lambda i: (0, 0)),
            pl.BlockSpec((1, 32), lambda i: (0, 0)),
            pl.BlockSpec((1, 1), lambda i: (0, 0), memory_space=pltpu.SMEM),
        ],
        out_specs=pl.BlockSpec((BLK,), lambda i: (i,)),
        out_shape=jax.ShapeDtypeStruct((B,), jnp.float32),
    )(uv, iv, w1u, w1i, b1, w2t, b2, w3, b3)


def kernel(user_indices, item_indices, user_table, item_table,
           W1, b1, W2, b2, W3, b3):
    tailu = user_table[NBU * 128:].reshape(16, 128)
    taili = item_table[NBI * 128:].reshape(8, 128)
    upk, ipk = _detile(user_table.T, item_table.T, tailu, taili)
    uvf, ivf = _gather(upk, ipk,
                       user_indices.astype(jnp.int32),
                       item_indices.astype(jnp.int32))
    uv = uvf.reshape(B, D)
    iv = ivf.reshape(B, D)
    w1u = W1[:, :D].T
    w1i = W1[:, D:].T
    return _mlp(uv, iv, w1u, w1i, b1.reshape(1, 64), W2.T,
                b2.reshape(1, 32), W3, b3.reshape(1, 1))
